# baseline (speedup 1.0000x reference)
import functools

import jax
import jax.numpy as jnp
from jax import lax
from jax.experimental import pallas as pl
from jax.experimental.pallas import tpu as pltpu
from jax.experimental.pallas import tpu_sc as plsc

EMBEDDING_DIM = 32
NUM_ROWS = 1000000
BATCH, SEQ = 16384, 26
B_TOTAL = BATCH * SEQ
NUM_CORES, NUM_SUBCORES = 2, 16
NW = NUM_CORES * NUM_SUBCORES

SLAB = 512
NSLAB = NUM_ROWS // SLAB
J_FULL = NSLAB // NW
EXTRA_C0 = J_FULL * NW * SLAB
MAIN_ROWS = EXTRA_C0 + SLAB

B_PER_W = B_TOTAL // NW
CHUNK = 1664
NCHUNK = B_PER_W // CHUNK
NBUF = 2

_mesh = plsc.VectorSubcoreMesh(core_axis_name="c", subcore_axis_name="s")


def _worker_id():
    return lax.axis_index("s") * NUM_CORES + lax.axis_index("c")


def _transpose_block(blk, tbuf, width, col0=0, out0=0):
    iota = lax.iota(jnp.int32, 16)
    iota32 = iota * 32
    zeros = jnp.zeros((16,), jnp.int32)
    ng = width // 16

    @plsc.parallel_loop(0, 32, unroll=8)
    def _(d):
        dvec = zeros + d
        for g in range(ng):
            v = plsc.load_gather(blk, [dvec, iota + (col0 + 16 * g)])
            plsc.store_scatter(tbuf, [iota32 + (out0 + 32 * 16 * g + d)], v)


@functools.partial(
    pl.kernel,
    out_type=jax.ShapeDtypeStruct((NUM_ROWS * EMBEDDING_DIM,), jnp.float32),
    mesh=_mesh,
    scratch_types=(
        [pltpu.VMEM((32, SLAB), jnp.float32) for _ in range(2)]
        + [pltpu.VMEM((SLAB * EMBEDDING_DIM,), jnp.float32) for _ in range(2)]
        + [pltpu.SemaphoreType.DMA for _ in range(4)]
    ),
    compiler_params=pltpu.CompilerParams(use_tc_tiling_on_sc=True,
                                         needs_layout_passes=False),
)
def _sc_repack(wt, tailp, out,
               blk0, blk1, tb0, tb1, si0, si1, so0, so1):
    blks, tbs = (blk0, blk1), (tb0, tb1)
    sis, sos = (si0, si1), (so0, so1)
    w = _worker_id()
    OUTW = SLAB * EMBEDDING_DIM

    def c0_of(j):
        return (j * NW + w) * SLAB

    def xpose_slab(blk, tbuf):
        for sub in range(SLAB // 128):
            _transpose_block(blk, tbuf, 128, col0=128 * sub, out0=4096 * sub)

    def start_in(j, slot):
        pltpu.async_copy(
            wt.at[:, pl.ds(c0_of(j), SLAB)], blks[slot], sis[slot])

    def wait_in(slot):
        pltpu.make_async_copy(
            wt.at[:, pl.ds(0, SLAB)], blks[slot], sis[slot]).wait()

    def start_out(j, slot):
        pltpu.async_copy(
            tbs[slot], out.at[pl.ds(c0_of(j) * EMBEDDING_DIM, OUTW)],
            sos[slot])

    def wait_out(slot):
        pltpu.make_async_copy(
            tbs[slot], out.at[pl.ds(0, OUTW)], sos[slot]).wait()

    start_in(0, 0)
    start_in(1, 1)

    def body(jj, _):
        for b in range(2):
            j = jj * 2 + b
            wait_in(b)

            @pl.when(jj > 0)
            def _():
                wait_out(b)

            xpose_slab(blks[b], tbs[b])

            @pl.when(j + 2 < J_FULL)
            def _():
                start_in(j + 2, b)

            start_out(j, b)
        return 0

    lax.fori_loop(0, (J_FULL - 1) // 2, body, 0)
    wait_in(0)
    wait_out(0)
    xpose_slab(blk0, tb0)
    start_out(J_FULL - 1, 0)
    wait_out(1)
    wait_out(0)

    @pl.when(w == 0)
    def _():
        pltpu.async_copy(wt.at[:, pl.ds(EXTRA_C0, SLAB)], blk1, si1)
        wait_in(1)
        xpose_slab(blk1, tb1)
        pltpu.async_copy(
            tb1, out.at[pl.ds(EXTRA_C0 * EMBEDDING_DIM, OUTW)], so1)
        wait_out(1)

    TAILN = (NUM_ROWS - MAIN_ROWS) * EMBEDDING_DIM

    @pl.when(w == 1)
    def _():
        pltpu.async_copy(tailp, blk1.at[:, pl.ds(0, 128)], si1)
        pltpu.make_async_copy(tailp, blk1.at[:, pl.ds(0, 128)], si1).wait()
        _transpose_block(blk1, tb1, 64)
        pltpu.async_copy(
            tb1.at[pl.ds(0, TAILN)],
            out.at[pl.ds(MAIN_ROWS * EMBEDDING_DIM, TAILN)], so1)
        pltpu.make_async_copy(
            tb1.at[pl.ds(0, TAILN)],
            out.at[pl.ds(MAIN_ROWS * EMBEDDING_DIM, TAILN)], so1).wait()


@functools.partial(
    pl.kernel,
    out_type=jax.ShapeDtypeStruct((B_TOTAL, EMBEDDING_DIM), jnp.float32),
    mesh=_mesh,
    scratch_types=(
        [pltpu.VMEM((CHUNK,), jnp.int32) for _ in range(NBUF)]
        + [pltpu.VMEM((CHUNK, EMBEDDING_DIM), jnp.float32) for _ in range(NBUF)]
        + [pltpu.SemaphoreType.DMA for _ in range(NBUF)]
    ),
    compiler_params=pltpu.CompilerParams(use_tc_tiling_on_sc=False),
)
def _sc_gather(idx_hbm, table_hbm, out_hbm,
               idx0, idx1, rows0, rows1, sem0, sem1):
    idx_bufs = (idx0, idx1)
    row_bufs = (rows0, rows1)
    sems = (sem0, sem1)
    base = _worker_id() * B_PER_W

    def start(g, slot):
        off = base + g * CHUNK
        pltpu.sync_copy(idx_hbm.at[pl.ds(off, CHUNK)], idx_bufs[slot])
        return pltpu.async_copy(table_hbm.at[idx_bufs[slot]], row_bufs[slot],
                                sems[slot])

    inflight = [None] * NBUF
    inflight[0] = start(0, 0)
    for g in range(NCHUNK):
        slot = g % NBUF
        nxt = (g + 1) % NBUF
        if g + 1 < NCHUNK:
            inflight[nxt] = start(g + 1, nxt)
        inflight[slot].wait()
        pltpu.sync_copy(row_bufs[slot],
                        out_hbm.at[pl.ds(base + g * CHUNK, CHUNK)])


def kernel(input_ids, weight):
    flat = input_ids.reshape(-1).astype(jnp.int32)
    wt = weight.T
    tailp = jnp.pad(wt[:, MAIN_ROWS:], ((0, 0), (0, 128 - (NUM_ROWS - MAIN_ROWS))))
    table = _sc_repack(wt, tailp).reshape(NUM_ROWS, EMBEDDING_DIM)
    out = _sc_gather(flat, table)
    return out.reshape(*input_ids.shape, EMBEDDING_DIM)


# speedup vs baseline: 1.6432x; 1.6432x over previous
import functools

import jax
import jax.numpy as jnp
from jax import lax
from jax.experimental import pallas as pl
from jax.experimental.pallas import tpu as pltpu
from jax.experimental.pallas import tpu_sc as plsc

EMBEDDING_DIM = 32
NUM_ROWS = 1000000
BATCH, SEQ = 16384, 26
B_TOTAL = BATCH * SEQ
NUM_CORES, NUM_SUBCORES = 2, 16
NW = NUM_CORES * NUM_SUBCORES

SLAB = 512
NSLAB = NUM_ROWS // SLAB
J_FULL = NSLAB // NW
EXTRA_C0 = J_FULL * NW * SLAB
MAIN_ROWS = EXTRA_C0 + SLAB

B_PER_W = B_TOTAL // NW
CHUNK = 1664
NCHUNK = B_PER_W // CHUNK
NBUF = 2

_mesh = plsc.VectorSubcoreMesh(core_axis_name="c", subcore_axis_name="s")


def _worker_id():
    return lax.axis_index("s") * NUM_CORES + lax.axis_index("c")


def _transpose_block(blk, tbuf, width, col0=0, out0=0):
    iota = lax.iota(jnp.int32, 16)
    rows = (iota, iota + 16)
    ng = width // 16

    @plsc.parallel_loop(0, 16, unroll=4)
    def _(k):
        m = (iota + k) & 15
        sm = m * 32 + iota
        for g in range(ng):
            for h in range(2):
                v = plsc.load_gather(blk, [rows[h], m + (col0 + 16 * g)])
                plsc.store_scatter(
                    tbuf, [sm + (out0 + 512 * g + 16 * h)], v)


@functools.partial(
    pl.kernel,
    out_type=jax.ShapeDtypeStruct((NUM_ROWS * EMBEDDING_DIM,), jnp.float32),
    mesh=_mesh,
    scratch_types=(
        [pltpu.VMEM((32, SLAB), jnp.float32) for _ in range(2)]
        + [pltpu.VMEM((SLAB * EMBEDDING_DIM,), jnp.float32) for _ in range(2)]
        + [pltpu.SemaphoreType.DMA for _ in range(4)]
    ),
    compiler_params=pltpu.CompilerParams(use_tc_tiling_on_sc=True,
                                         needs_layout_passes=False),
)
def _sc_repack(wt, tailp, out,
               blk0, blk1, tb0, tb1, si0, si1, so0, so1):
    blks, tbs = (blk0, blk1), (tb0, tb1)
    sis, sos = (si0, si1), (so0, so1)
    w = _worker_id()
    OUTW = SLAB * EMBEDDING_DIM

    def c0_of(j):
        return (j * NW + w) * SLAB

    def xpose_slab(blk, tbuf):
        for sub in range(SLAB // 128):
            _transpose_block(blk, tbuf, 128, col0=128 * sub, out0=4096 * sub)

    def start_in(j, slot):
        pltpu.async_copy(
            wt.at[:, pl.ds(c0_of(j), SLAB)], blks[slot], sis[slot])

    def wait_in(slot):
        pltpu.make_async_copy(
            wt.at[:, pl.ds(0, SLAB)], blks[slot], sis[slot]).wait()

    def start_out(j, slot):
        pltpu.async_copy(
            tbs[slot], out.at[pl.ds(c0_of(j) * EMBEDDING_DIM, OUTW)],
            sos[slot])

    def wait_out(slot):
        pltpu.make_async_copy(
            tbs[slot], out.at[pl.ds(0, OUTW)], sos[slot]).wait()

    start_in(0, 0)
    start_in(1, 1)

    def body(jj, _):
        for b in range(2):
            j = jj * 2 + b
            wait_in(b)

            @pl.when(jj > 0)
            def _():
                wait_out(b)

            xpose_slab(blks[b], tbs[b])

            @pl.when(j + 2 < J_FULL)
            def _():
                start_in(j + 2, b)

            start_out(j, b)
        return 0

    lax.fori_loop(0, (J_FULL - 1) // 2, body, 0)
    wait_in(0)
    wait_out(0)
    xpose_slab(blk0, tb0)
    start_out(J_FULL - 1, 0)
    wait_out(1)
    wait_out(0)

    @pl.when(w == 0)
    def _():
        pltpu.async_copy(wt.at[:, pl.ds(EXTRA_C0, SLAB)], blk1, si1)
        wait_in(1)
        xpose_slab(blk1, tb1)
        pltpu.async_copy(
            tb1, out.at[pl.ds(EXTRA_C0 * EMBEDDING_DIM, OUTW)], so1)
        wait_out(1)

    TAILN = (NUM_ROWS - MAIN_ROWS) * EMBEDDING_DIM

    @pl.when(w == 1)
    def _():
        pltpu.async_copy(tailp, blk1.at[:, pl.ds(0, 128)], si1)
        pltpu.make_async_copy(tailp, blk1.at[:, pl.ds(0, 128)], si1).wait()
        _transpose_block(blk1, tb1, 64)
        pltpu.async_copy(
            tb1.at[pl.ds(0, TAILN)],
            out.at[pl.ds(MAIN_ROWS * EMBEDDING_DIM, TAILN)], so1)
        pltpu.make_async_copy(
            tb1.at[pl.ds(0, TAILN)],
            out.at[pl.ds(MAIN_ROWS * EMBEDDING_DIM, TAILN)], so1).wait()


@functools.partial(
    pl.kernel,
    out_type=jax.ShapeDtypeStruct((B_TOTAL, EMBEDDING_DIM), jnp.float32),
    mesh=_mesh,
    scratch_types=(
        [pltpu.VMEM((CHUNK,), jnp.int32) for _ in range(NBUF)]
        + [pltpu.VMEM((CHUNK, EMBEDDING_DIM), jnp.float32) for _ in range(NBUF)]
        + [pltpu.SemaphoreType.DMA for _ in range(NBUF)]
    ),
    compiler_params=pltpu.CompilerParams(use_tc_tiling_on_sc=False),
)
def _sc_gather(idx_hbm, table_hbm, out_hbm,
               idx0, idx1, rows0, rows1, sem0, sem1):
    idx_bufs = (idx0, idx1)
    row_bufs = (rows0, rows1)
    sems = (sem0, sem1)
    base = _worker_id() * B_PER_W

    def start(g, slot):
        off = base + g * CHUNK
        pltpu.sync_copy(idx_hbm.at[pl.ds(off, CHUNK)], idx_bufs[slot])
        return pltpu.async_copy(table_hbm.at[idx_bufs[slot]], row_bufs[slot],
                                sems[slot])

    inflight = [None] * NBUF
    inflight[0] = start(0, 0)
    for g in range(NCHUNK):
        slot = g % NBUF
        nxt = (g + 1) % NBUF
        if g + 1 < NCHUNK:
            inflight[nxt] = start(g + 1, nxt)
        inflight[slot].wait()
        pltpu.sync_copy(row_bufs[slot],
                        out_hbm.at[pl.ds(base + g * CHUNK, CHUNK)])


def kernel(input_ids, weight):
    flat = input_ids.reshape(-1).astype(jnp.int32)
    wt = weight.T
    tailp = jnp.pad(wt[:, MAIN_ROWS:], ((0, 0), (0, 128 - (NUM_ROWS - MAIN_ROWS))))
    table = _sc_repack(wt, tailp).reshape(NUM_ROWS, EMBEDDING_DIM)
    out = _sc_gather(flat, table)
    return out.reshape(*input_ids.shape, EMBEDDING_DIM)


# speedup vs baseline: 3.2157x; 1.9569x over previous
import functools

import jax
import jax.numpy as jnp
from jax import lax
from jax.experimental import pallas as pl
from jax.experimental.pallas import tpu as pltpu
from jax.experimental.pallas import tpu_sc as plsc

EMBEDDING_DIM = 32
NUM_ROWS = 1000000
BATCH, SEQ = 16384, 26
B_TOTAL = BATCH * SEQ
NUM_CORES, NUM_SUBCORES = 2, 16
NW = NUM_CORES * NUM_SUBCORES

SLAB = 512
NSLAB = NUM_ROWS // SLAB
J_FULL = NSLAB // NW
EXTRA_C0 = J_FULL * NW * SLAB
MAIN_ROWS = EXTRA_C0 + SLAB

CH = 512
KPS = BATCH // CH
NCHUNK_TOT = SEQ * KPS
T_PER_W = NCHUNK_TOT // NW
CHW = CH * EMBEDDING_DIM

_mesh = plsc.VectorSubcoreMesh(core_axis_name="c", subcore_axis_name="s")


def _worker_id():
    return lax.axis_index("s") * NUM_CORES + lax.axis_index("c")


def _transpose_block(blk, tbuf, width, col0=0, out0=0):
    iota = lax.iota(jnp.int32, 16)
    rows = (iota, iota + 16)
    ng = width // 16

    @plsc.parallel_loop(0, 16, unroll=4)
    def _(k):
        m = (iota + k) & 15
        sm = m * 32 + iota
        for g in range(ng):
            for h in range(2):
                v = plsc.load_gather(blk, [rows[h], m + (col0 + 16 * g)])
                plsc.store_scatter(
                    tbuf, [sm + (out0 + 512 * g + 16 * h)], v)


@functools.partial(
    pl.kernel,
    out_type=jax.ShapeDtypeStruct((NUM_ROWS * EMBEDDING_DIM,), jnp.float32),
    mesh=_mesh,
    scratch_types=(
        [pltpu.VMEM((32, SLAB), jnp.float32) for _ in range(2)]
        + [pltpu.VMEM((SLAB * EMBEDDING_DIM,), jnp.float32) for _ in range(2)]
        + [pltpu.SemaphoreType.DMA for _ in range(4)]
    ),
    compiler_params=pltpu.CompilerParams(use_tc_tiling_on_sc=True,
                                         needs_layout_passes=False),
)
def _sc_repack(wt, tailp, out,
               blk0, blk1, tb0, tb1, si0, si1, so0, so1):
    blks, tbs = (blk0, blk1), (tb0, tb1)
    sis, sos = (si0, si1), (so0, so1)
    w = _worker_id()
    OUTW = SLAB * EMBEDDING_DIM

    def c0_of(j):
        return (j * NW + w) * SLAB

    def xpose_slab(blk, tbuf):
        for sub in range(SLAB // 128):
            _transpose_block(blk, tbuf, 128, col0=128 * sub, out0=4096 * sub)

    def start_in(j, slot):
        pltpu.async_copy(
            wt.at[:, pl.ds(c0_of(j), SLAB)], blks[slot], sis[slot])

    def wait_in(slot):
        pltpu.make_async_copy(
            wt.at[:, pl.ds(0, SLAB)], blks[slot], sis[slot]).wait()

    def start_out(j, slot):
        pltpu.async_copy(
            tbs[slot], out.at[pl.ds(c0_of(j) * EMBEDDING_DIM, OUTW)],
            sos[slot])

    def wait_out(slot):
        pltpu.make_async_copy(
            tbs[slot], out.at[pl.ds(0, OUTW)], sos[slot]).wait()

    start_in(0, 0)
    start_in(1, 1)

    def body(jj, _):
        for b in range(2):
            j = jj * 2 + b
            wait_in(b)

            @pl.when(jj > 0)
            def _():
                wait_out(b)

            xpose_slab(blks[b], tbs[b])

            @pl.when(j + 2 < J_FULL)
            def _():
                start_in(j + 2, b)

            start_out(j, b)
        return 0

    lax.fori_loop(0, (J_FULL - 1) // 2, body, 0)
    wait_in(0)
    wait_out(0)
    xpose_slab(blk0, tb0)
    start_out(J_FULL - 1, 0)
    wait_out(1)
    wait_out(0)

    @pl.when(w == 0)
    def _():
        pltpu.async_copy(wt.at[:, pl.ds(EXTRA_C0, SLAB)], blk1, si1)
        wait_in(1)
        xpose_slab(blk1, tb1)
        pltpu.async_copy(
            tb1, out.at[pl.ds(EXTRA_C0 * EMBEDDING_DIM, OUTW)], so1)
        wait_out(1)

    TAILN = (NUM_ROWS - MAIN_ROWS) * EMBEDDING_DIM

    @pl.when(w == 1)
    def _():
        pltpu.async_copy(tailp, blk1.at[:, pl.ds(0, 128)], si1)
        pltpu.make_async_copy(tailp, blk1.at[:, pl.ds(0, 128)], si1).wait()
        _transpose_block(blk1, tb1, 64)
        pltpu.async_copy(
            tb1.at[pl.ds(0, TAILN)],
            out.at[pl.ds(MAIN_ROWS * EMBEDDING_DIM, TAILN)], so1)
        pltpu.make_async_copy(
            tb1.at[pl.ds(0, TAILN)],
            out.at[pl.ds(MAIN_ROWS * EMBEDDING_DIM, TAILN)], so1).wait()


@functools.partial(
    pl.kernel,
    out_type=jax.ShapeDtypeStruct((B_TOTAL * EMBEDDING_DIM,), jnp.float32),
    mesh=_mesh,
    scratch_types=(
        [pltpu.VMEM((CH,), jnp.int32) for _ in range(2)]
        + [pltpu.VMEM((CH, EMBEDDING_DIM), jnp.float32) for _ in range(2)]
        + [pltpu.VMEM((CHW,), jnp.float32) for _ in range(2)]
        + [pltpu.SemaphoreType.DMA for _ in range(4)]
    ),
    compiler_params=pltpu.CompilerParams(use_tc_tiling_on_sc=False,
                                         needs_layout_passes=False),
)
def _sc_gather(idsT_hbm, table_hbm, out_hbm,
               ix0, ix1, r0, r1, t0, t1, sg0, sg1, so0, so1):
    ixs, rws, tchs = (ix0, ix1), (r0, r1), (t0, t1)
    sgs, sos = (sg0, sg1), (so0, so1)
    w = _worker_id()
    iota = lax.iota(jnp.int32, 16)

    def start_gather(tt, slot):
        cid = w * T_PER_W + tt
        s, k = cid // KPS, cid % KPS
        pltpu.sync_copy(idsT_hbm.at[pl.ds(s * BATCH + k * CH, CH)], ixs[slot])
        pltpu.async_copy(table_hbm.at[ixs[slot]], rws[slot], sgs[slot])

    def wait_gather(slot):
        pltpu.make_async_copy(
            table_hbm.at[ixs[slot]], rws[slot], sgs[slot]).wait()

    def start_out(tt, slot):
        cid = w * T_PER_W + tt
        s, k = cid // KPS, cid % KPS
        for R in range(4):
            off = (s * 4 + R) * (128 * 1024) + k * (4 * 1024)
            pltpu.async_copy(tchs[slot].at[pl.ds(R * 4096, 4096)],
                             out_hbm.at[pl.ds(off, 4096)], sos[slot])

    def wait_out(slot):
        pltpu.make_async_copy(
            tchs[slot], out_hbm.at[pl.ds(0, CHW)], sos[slot]).wait()

    def xpose_chunk(rows, tch):
        @plsc.parallel_loop(0, 16, unroll=4)
        def _(k2):
            m = (iota + k2) & 15
            md = (m // 8) * 4096 + (m % 8) * 128 + iota
            mh = (m, m + 16)
            for bg in range(CH // 16):
                for h in range(2):
                    v = plsc.load_gather(rows, [iota + 16 * bg, mh[h]])
                    dest = md + (8192 * h + (bg // 8) * 1024 + 16 * (bg % 8))
                    plsc.store_scatter(tch, [dest], v)

    start_gather(0, 0)
    start_gather(1, 1)

    def body(jj, _):
        for b in range(2):
            tt = jj * 2 + b
            wait_gather(b)

            @pl.when(jj > 0)
            def _():
                wait_out(b)

            xpose_chunk(rws[b], tchs[b])

            @pl.when(tt + 2 < T_PER_W)
            def _():
                start_gather(tt + 2, b)

            start_out(tt, b)
        return 0

    lax.fori_loop(0, T_PER_W // 2, body, 0)
    wait_out(0)
    wait_out(1)


def kernel(input_ids, weight):
    flat_t = input_ids.T.reshape(-1).astype(jnp.int32)
    wt = weight.T
    tailp = jnp.pad(wt[:, MAIN_ROWS:], ((0, 0), (0, 128 - (NUM_ROWS - MAIN_ROWS))))
    table = _sc_repack(wt, tailp).reshape(NUM_ROWS, EMBEDDING_DIM)
    out = _sc_gather(flat_t, table)
    out5d = out.reshape(SEQ, 4, 128, 8, 128)
    return out5d.transpose(2, 4, 0, 1, 3).reshape(BATCH, SEQ, EMBEDDING_DIM)


# speedup vs baseline: 3.2505x; 1.0108x over previous
import functools

import jax
import jax.numpy as jnp
from jax import lax
from jax.experimental import pallas as pl
from jax.experimental.pallas import tpu as pltpu
from jax.experimental.pallas import tpu_sc as plsc

EMBEDDING_DIM = 32
NUM_ROWS = 1000000
BATCH, SEQ = 16384, 26
B_TOTAL = BATCH * SEQ
NUM_CORES, NUM_SUBCORES = 2, 16
NW = NUM_CORES * NUM_SUBCORES

SLAB = 512
NSLAB = NUM_ROWS // SLAB
J_FULL = NSLAB // NW
EXTRA_C0 = J_FULL * NW * SLAB
MAIN_ROWS = EXTRA_C0 + SLAB

CH = 512
KPS = BATCH // CH
NCHUNK_TOT = SEQ * KPS
T_PER_W = NCHUNK_TOT // NW
CHW = CH * EMBEDDING_DIM

_mesh = plsc.VectorSubcoreMesh(core_axis_name="c", subcore_axis_name="s")


def _worker_id():
    return lax.axis_index("s") * NUM_CORES + lax.axis_index("c")


def _transpose_block(blk, tbuf, width, col0=0, out0=0):
    iota = lax.iota(jnp.int32, 16)
    rows = (iota, iota + 16)
    ng = width // 16

    @plsc.parallel_loop(0, 16, unroll=4)
    def _(k):
        m = (iota + k) & 15
        sm = m * 32 + iota
        for g in range(ng):
            for h in range(2):
                v = plsc.load_gather(blk, [rows[h], m + (col0 + 16 * g)])
                plsc.store_scatter(
                    tbuf, [sm + (out0 + 512 * g + 16 * h)], v)


@functools.partial(
    pl.kernel,
    out_type=jax.ShapeDtypeStruct((NUM_ROWS * EMBEDDING_DIM,), jnp.float32),
    mesh=_mesh,
    scratch_types=(
        [pltpu.VMEM((32, SLAB), jnp.float32) for _ in range(3)]
        + [pltpu.VMEM((SLAB * EMBEDDING_DIM,), jnp.float32) for _ in range(3)]
        + [pltpu.SemaphoreType.DMA for _ in range(6)]
    ),
    compiler_params=pltpu.CompilerParams(use_tc_tiling_on_sc=True,
                                         needs_layout_passes=False),
)
def _sc_repack(wt, tailp, out,
               blk0, blk1, blk2, tb0, tb1, tb2,
               si0, si1, si2, so0, so1, so2):
    blks, tbs = (blk0, blk1, blk2), (tb0, tb1, tb2)
    sis, sos = (si0, si1, si2), (so0, so1, so2)
    w = _worker_id()
    OUTW = SLAB * EMBEDDING_DIM

    def c0_of(j):
        return (j * NW + w) * SLAB

    def xpose_slab(blk, tbuf):
        for sub in range(SLAB // 128):
            _transpose_block(blk, tbuf, 128, col0=128 * sub, out0=4096 * sub)

    def start_in(j, slot):
        pltpu.async_copy(
            wt.at[:, pl.ds(c0_of(j), SLAB)], blks[slot], sis[slot])

    def wait_in(slot):
        pltpu.make_async_copy(
            wt.at[:, pl.ds(0, SLAB)], blks[slot], sis[slot]).wait()

    def start_out(j, slot):
        pltpu.async_copy(
            tbs[slot], out.at[pl.ds(c0_of(j) * EMBEDDING_DIM, OUTW)],
            sos[slot])

    def wait_out(slot):
        pltpu.make_async_copy(
            tbs[slot], out.at[pl.ds(0, OUTW)], sos[slot]).wait()

    start_in(0, 0)
    start_in(1, 1)
    start_in(2, 2)

    def body(jj, _):
        for b in range(3):
            j = jj * 3 + b
            wait_in(b)

            @pl.when(jj > 0)
            def _():
                wait_out(b)

            xpose_slab(blks[b], tbs[b])

            @pl.when(j + 3 < J_FULL)
            def _():
                start_in(j + 3, b)

            start_out(j, b)
        return 0

    lax.fori_loop(0, (J_FULL - 1) // 3, body, 0)
    wait_in(0)
    wait_out(0)
    xpose_slab(blk0, tb0)
    start_out(J_FULL - 1, 0)
    wait_out(1)
    wait_out(2)
    wait_out(0)

    @pl.when(w == 0)
    def _():
        pltpu.async_copy(wt.at[:, pl.ds(EXTRA_C0, SLAB)], blk1, si1)
        wait_in(1)
        xpose_slab(blk1, tb1)
        pltpu.async_copy(
            tb1, out.at[pl.ds(EXTRA_C0 * EMBEDDING_DIM, OUTW)], so1)
        wait_out(1)

    TAILN = (NUM_ROWS - MAIN_ROWS) * EMBEDDING_DIM

    @pl.when(w == 1)
    def _():
        pltpu.async_copy(tailp, blk1.at[:, pl.ds(0, 128)], si1)
        pltpu.make_async_copy(tailp, blk1.at[:, pl.ds(0, 128)], si1).wait()
        _transpose_block(blk1, tb1, 64)
        pltpu.async_copy(
            tb1.at[pl.ds(0, TAILN)],
            out.at[pl.ds(MAIN_ROWS * EMBEDDING_DIM, TAILN)], so1)
        pltpu.make_async_copy(
            tb1.at[pl.ds(0, TAILN)],
            out.at[pl.ds(MAIN_ROWS * EMBEDDING_DIM, TAILN)], so1).wait()


@functools.partial(
    pl.kernel,
    out_type=jax.ShapeDtypeStruct((B_TOTAL * EMBEDDING_DIM,), jnp.float32),
    mesh=_mesh,
    scratch_types=(
        [pltpu.VMEM((CH,), jnp.int32) for _ in range(2)]
        + [pltpu.VMEM((CH, EMBEDDING_DIM), jnp.float32) for _ in range(2)]
        + [pltpu.VMEM((CHW,), jnp.float32) for _ in range(2)]
        + [pltpu.SemaphoreType.DMA for _ in range(4)]
    ),
    compiler_params=pltpu.CompilerParams(use_tc_tiling_on_sc=False,
                                         needs_layout_passes=False),
)
def _sc_gather(idsT_hbm, table_hbm, out_hbm,
               ix0, ix1, r0, r1, t0, t1, sg0, sg1, so0, so1):
    ixs, rws, tchs = (ix0, ix1), (r0, r1), (t0, t1)
    sgs, sos = (sg0, sg1), (so0, so1)
    w = _worker_id()
    iota = lax.iota(jnp.int32, 16)

    def start_gather(tt, slot):
        cid = w * T_PER_W + tt
        s, k = cid // KPS, cid % KPS
        pltpu.sync_copy(idsT_hbm.at[pl.ds(s * BATCH + k * CH, CH)], ixs[slot])
        pltpu.async_copy(table_hbm.at[ixs[slot]], rws[slot], sgs[slot])

    def wait_gather(slot):
        pltpu.make_async_copy(
            table_hbm.at[ixs[slot]], rws[slot], sgs[slot]).wait()

    def start_out(tt, slot):
        cid = w * T_PER_W + tt
        s, k = cid // KPS, cid % KPS
        for R in range(4):
            off = (s * 4 + R) * (128 * 1024) + k * (4 * 1024)
            pltpu.async_copy(tchs[slot].at[pl.ds(R * 4096, 4096)],
                             out_hbm.at[pl.ds(off, 4096)], sos[slot])

    def wait_out(slot):
        pltpu.make_async_copy(
            tchs[slot], out_hbm.at[pl.ds(0, CHW)], sos[slot]).wait()

    def xpose_chunk(rows, tch):
        @plsc.parallel_loop(0, 16, unroll=4)
        def _(k2):
            m = (iota + k2) & 15
            md = (m // 8) * 4096 + (m % 8) * 128 + iota
            mh = (m, m + 16)
            for bg in range(CH // 16):
                for h in range(2):
                    v = plsc.load_gather(rows, [iota + 16 * bg, mh[h]])
                    dest = md + (8192 * h + (bg // 8) * 1024 + 16 * (bg % 8))
                    plsc.store_scatter(tch, [dest], v)

    start_gather(0, 0)
    start_gather(1, 1)

    def body(jj, _):
        for b in range(2):
            tt = jj * 2 + b
            wait_gather(b)

            @pl.when(jj > 0)
            def _():
                wait_out(b)

            xpose_chunk(rws[b], tchs[b])

            @pl.when(tt + 2 < T_PER_W)
            def _():
                start_gather(tt + 2, b)

            start_out(tt, b)
        return 0

    lax.fori_loop(0, T_PER_W // 2, body, 0)
    wait_out(0)
    wait_out(1)


def kernel(input_ids, weight):
    flat_t = input_ids.T.reshape(-1).astype(jnp.int32)
    wt = weight.T
    tailp = jnp.pad(wt[:, MAIN_ROWS:], ((0, 0), (0, 128 - (NUM_ROWS - MAIN_ROWS))))
    table = _sc_repack(wt, tailp).reshape(NUM_ROWS, EMBEDDING_DIM)
    out = _sc_gather(flat_t, table)
    out5d = out.reshape(SEQ, 4, 128, 8, 128)
    return out5d.transpose(2, 4, 0, 1, 3).reshape(BATCH, SEQ, EMBEDDING_DIM)
